# Initial kernel scaffold; baseline (speedup 1.0000x reference)
import jax
import jax.numpy as jnp
from jax.experimental import pallas as pl


def kernel(x, edge_index, W1, b1, W2, b2):
    raise NotImplementedError("write your pallas kernel here")


# speedup vs baseline: 38.8695x; 38.8695x over previous
import jax
import jax.numpy as jnp
from jax import lax
from jax.experimental import pallas as pl
from jax.experimental.pallas import tpu as pltpu
from jax.experimental.pallas import tpu_sc as plsc

N = 10000
E = 320000
D_IN = 128
D_HID = 16
D_OUT = 40

NROWS = 10240
PAD_IDX = 10000
CHUNK = 128
CPT32 = 80
CPT16 = 2 * CPT32
EROWS = 32 * CPT32
E_PAD = EROWS * CHUNK
RPT = NROWS // 16

_mesh = plsc.VectorSubcoreMesh(core_axis_name="c", subcore_axis_name="s")


def _deg_body(dst_hbm, dis_hbm, idx_v, acc_v, part_sh, col_v, out_v):
    c = lax.axis_index("c")
    s = lax.axis_index("s")

    @pl.when(c == 0)
    def _():
        pltpu.sync_copy(dst_hbm.at[pl.ds(s * CPT16, CPT16)], idx_v)

        zeros16 = jnp.zeros((16,), jnp.float32)

        def zloop(i, carry):
            acc_v[pl.ds(i * 16, 16)] = zeros16
            return carry

        lax.fori_loop(0, NROWS // 16, zloop, None)

        ones16 = jnp.full((16,), 1.0, jnp.float32)

        def cloop(r, carry):
            for k in range(CHUNK // 16):
                idx = idx_v[r, pl.ds(k * 16, 16)]
                plsc.addupdate_scatter(acc_v, [idx], ones16)
            return carry

        lax.fori_loop(0, CPT16, cloop, None)

        pltpu.sync_copy(acc_v, part_sh.at[s])
        plsc.subcore_barrier()
        for r in range(16):
            pltpu.sync_copy(part_sh.at[r, pl.ds(s * RPT, RPT)], col_v.at[r])

        def sloop(i, carry):
            deg = jnp.full((16,), 1.0, jnp.float32)
            for r in range(16):
                deg = deg + col_v[r, pl.ds(i * 16, 16)]
            bi = plsc.bitcast(deg, jnp.int32)
            bi = 0x5F3759DF - lax.shift_right_arithmetic(bi, 1)
            y = plsc.bitcast(bi, jnp.float32)
            for _n in range(3):
                y = y * (1.5 - 0.5 * deg * y * y)
            out_v[pl.ds(i * 16, 16)] = y
            return carry

        lax.fori_loop(0, RPT // 16, sloop, None)
        pltpu.sync_copy(out_v, dis_hbm.at[pl.ds(s * RPT, RPT)])


_deg_call = pl.kernel(
    _deg_body,
    out_type=jax.ShapeDtypeStruct((NROWS,), jnp.float32),
    mesh=_mesh,
    compiler_params=pltpu.CompilerParams(needs_layout_passes=False, use_tc_tiling_on_sc=False),
    scratch_types=[
        pltpu.VMEM((CPT16, CHUNK), jnp.int32),
        pltpu.VMEM((NROWS,), jnp.float32),
        pltpu.VMEM_SHARED((16, NROWS), jnp.float32),
        pltpu.VMEM((16, RPT), jnp.float32),
        pltpu.VMEM((RPT,), jnp.float32),
    ],
)


def _agg_body(h_hbm, src_hbm, dst_hbm, out_hbm,
              sidx, didx, rows0, rows1, stage, acc_sh, sem0, sem1):
    c = lax.axis_index("c")
    s = lax.axis_index("s")
    w = c * 16 + s
    pltpu.sync_copy(src_hbm.at[pl.ds(w * CPT32, CPT32)], sidx)
    pltpu.sync_copy(dst_hbm.at[pl.ds(w * CPT32, CPT32)], didx)

    zeros16 = jnp.zeros((16,), jnp.float32)

    def zloop(i, carry):
        stage[i] = zeros16
        return carry

    lax.fori_loop(0, RPT, zloop, None)
    pltpu.sync_copy(stage, acc_sh.at[pl.ds(s * RPT, RPT)])
    plsc.subcore_barrier()

    def fire(j, buf, sem):
        pltpu.async_copy(h_hbm.at[sidx.at[j]], buf, sem)

    def drain_scatter(j, buf, sem):
        pltpu.make_async_copy(h_hbm.at[sidx.at[j]], buf, sem).wait()
        pltpu.sync_copy(buf, acc_sh.at[didx.at[j]], add=True)

    fire(0, rows0, sem0)
    fire(1, rows1, sem1)

    def pair(p, carry):
        j0 = 2 * p
        drain_scatter(j0, rows0, sem0)
        fire(j0 + 2, rows0, sem0)
        drain_scatter(j0 + 1, rows1, sem1)
        fire(j0 + 3, rows1, sem1)
        return carry

    lax.fori_loop(0, CPT32 // 2 - 1, pair, None)
    drain_scatter(CPT32 - 2, rows0, sem0)
    drain_scatter(CPT32 - 1, rows1, sem1)

    plsc.subcore_barrier()
    pltpu.sync_copy(acc_sh.at[pl.ds(s * RPT, RPT)], stage)
    pltpu.sync_copy(stage, out_hbm.at[c, pl.ds(s * RPT, RPT)])


_agg_call = pl.kernel(
    _agg_body,
    out_type=jax.ShapeDtypeStruct((2, NROWS, D_HID), jnp.float32),
    mesh=_mesh,
    compiler_params=pltpu.CompilerParams(needs_layout_passes=False, use_tc_tiling_on_sc=False),
    scratch_types=[
        pltpu.VMEM((CPT32, CHUNK), jnp.int32),
        pltpu.VMEM((CPT32, CHUNK), jnp.int32),
        pltpu.VMEM((CHUNK, D_HID), jnp.float32),
        pltpu.VMEM((CHUNK, D_HID), jnp.float32),
        pltpu.VMEM((RPT, D_HID), jnp.float32),
        pltpu.VMEM_SHARED((NROWS, D_HID), jnp.float32),
        pltpu.SemaphoreType.DMA,
        pltpu.SemaphoreType.DMA,
    ],
)


def _k2_body(x_ref, w_ref, dis_ref, o_ref):
    h = jnp.dot(x_ref[...], w_ref[...], preferred_element_type=jnp.float32)
    o_ref[...] = h * dis_ref[...]


def _k4_body(parts_ref, h_ref, dis_ref, b_ref, o_ref):
    agg = parts_ref[0] + parts_ref[1] + h_ref[...]
    z = jnp.maximum(agg * dis_ref[...] + b_ref[...], 0.0)
    o_ref[...] = z * dis_ref[...]


def _k6_body(parts_ref, z_ref, dis_ref, w_ref, b_ref, o_ref):
    u = (parts_ref[0] + parts_ref[1] + z_ref[...]) * dis_ref[...]
    o = jnp.dot(u, w_ref[...], preferred_element_type=jnp.float32) + b_ref[...]
    m = jnp.max(o, axis=1, keepdims=True)
    e = o - m
    lse = jnp.log(jnp.sum(jnp.exp(e), axis=1, keepdims=True))
    o_ref[...] = e - lse


_k2_call = pl.pallas_call(
    _k2_body, out_shape=jax.ShapeDtypeStruct((NROWS, D_HID), jnp.float32))
_k4_call = pl.pallas_call(
    _k4_body, out_shape=jax.ShapeDtypeStruct((NROWS, D_HID), jnp.float32))
_k6_call = pl.pallas_call(
    _k6_body, out_shape=jax.ShapeDtypeStruct((NROWS, D_OUT), jnp.float32))


@jax.jit
def kernel(x, edge_index, W1, b1, W2, b2):
    f32 = jnp.float32
    src = edge_index[0]
    dst = edge_index[1]
    pad = jnp.full((E_PAD - E,), PAD_IDX, dtype=jnp.int32)
    srcp = jnp.concatenate([src, pad]).reshape(EROWS, CHUNK)
    dstp = jnp.concatenate([dst, pad]).reshape(EROWS, CHUNK)
    xp = jnp.concatenate([x, jnp.zeros((NROWS - N, D_IN), f32)], axis=0)

    dis = _deg_call(dstp)
    dis2 = dis.reshape(NROWS, 1)
    h1p = _k2_call(xp, W1, dis2)
    parts1 = _agg_call(h1p, srcp, dstp)
    zp = _k4_call(parts1, h1p, dis2, b1.reshape(1, D_HID))
    parts2 = _agg_call(zp, srcp, dstp)
    out = _k6_call(parts2, zp, dis2, W2, b2.reshape(1, D_OUT))
    return out[:N]


# speedup vs baseline: 57.1170x; 1.4695x over previous
import jax
import jax.numpy as jnp
from jax import lax
from jax.experimental import pallas as pl
from jax.experimental.pallas import tpu as pltpu
from jax.experimental.pallas import tpu_sc as plsc

N = 10000
E = 320000
D_IN = 128
D_HID = 16
D_OUT = 40

NROWS = 10240
PAD_IDX = 10000
CHUNK = 128
CPT32 = 80
CPT16 = 2 * CPT32
EROWS = 32 * CPT32
E_PAD = EROWS * CHUNK
RPT = NROWS // 16

_mesh = plsc.VectorSubcoreMesh(core_axis_name="c", subcore_axis_name="s")


def _deg_body(dst_hbm, dis_hbm, idx_v, acc_v, part_sh, col_v, out_v):
    c = lax.axis_index("c")
    s = lax.axis_index("s")

    @pl.when(c == 0)
    def _():
        pltpu.sync_copy(dst_hbm.at[pl.ds(s * CPT16, CPT16)], idx_v)

        zeros16 = jnp.zeros((16,), jnp.float32)

        def zloop(i, carry):
            acc_v[pl.ds(i * 16, 16)] = zeros16
            return carry

        lax.fori_loop(0, NROWS // 16, zloop, None)

        ones16 = jnp.full((16,), 1.0, jnp.float32)

        def cloop(r, carry):
            for k in range(CHUNK // 16):
                idx = idx_v[r, pl.ds(k * 16, 16)]
                plsc.addupdate_scatter(acc_v, [idx], ones16)
            return carry

        lax.fori_loop(0, CPT16, cloop, None)

        pltpu.sync_copy(acc_v, part_sh.at[s])
        plsc.subcore_barrier()
        for r in range(16):
            pltpu.sync_copy(part_sh.at[r, pl.ds(s * RPT, RPT)], col_v.at[r])

        def sloop(i, carry):
            deg = jnp.full((16,), 1.0, jnp.float32)
            for r in range(16):
                deg = deg + col_v[r, pl.ds(i * 16, 16)]
            bi = plsc.bitcast(deg, jnp.int32)
            bi = 0x5F3759DF - lax.shift_right_arithmetic(bi, 1)
            y = plsc.bitcast(bi, jnp.float32)
            for _n in range(3):
                y = y * (1.5 - 0.5 * deg * y * y)
            out_v[pl.ds(i * 16, 16)] = y
            return carry

        lax.fori_loop(0, RPT // 16, sloop, None)
        pltpu.sync_copy(out_v, dis_hbm.at[pl.ds(s * RPT, RPT)])


_deg_call = pl.kernel(
    _deg_body,
    out_type=jax.ShapeDtypeStruct((NROWS,), jnp.float32),
    mesh=_mesh,
    compiler_params=pltpu.CompilerParams(needs_layout_passes=False, use_tc_tiling_on_sc=False),
    scratch_types=[
        pltpu.VMEM((CPT16, CHUNK), jnp.int32),
        pltpu.VMEM((NROWS,), jnp.float32),
        pltpu.VMEM_SHARED((16, NROWS), jnp.float32),
        pltpu.VMEM((16, RPT), jnp.float32),
        pltpu.VMEM((RPT,), jnp.float32),
    ],
)


NBUF = 4


def _make_agg(scale):
    def body(h_hbm, dis_hbm, src_hbm, dst_hbm, out_hbm,
             sidx, didx, b0, b1, b2, b3, stage, disv, table_sh, acc_sh,
             g0, g1, g2, g3, s0, s1, s2, s3):
        c = lax.axis_index("c")
        s = lax.axis_index("s")
        w = c * 16 + s
        pltpu.sync_copy(src_hbm.at[pl.ds(w * CPT32, CPT32)], sidx)
        pltpu.sync_copy(dst_hbm.at[pl.ds(w * CPT32, CPT32)], didx)

        pltpu.sync_copy(h_hbm.at[pl.ds(s * RPT, RPT)], stage)
        if scale:
            pltpu.sync_copy(dis_hbm.at[pl.ds(s * RPT, RPT)], disv)

            def scl(i, carry):
                d = plsc.load_gather(disv, [jnp.full((16,), i, jnp.int32)])
                stage[i] = stage[i] * d
                return carry

            lax.fori_loop(0, RPT, scl, None)
        pltpu.sync_copy(stage, table_sh.at[pl.ds(s * RPT, RPT)])

        zeros16 = jnp.zeros((16,), jnp.float32)

        def zloop(i, carry):
            stage[i] = zeros16
            return carry

        lax.fori_loop(0, RPT, zloop, None)
        pltpu.sync_copy(stage, acc_sh.at[pl.ds(s * RPT, RPT)])
        plsc.subcore_barrier()

        bufs = (b0, b1, b2, b3)
        gsems = (g0, g1, g2, g3)
        ssems = (s0, s1, s2, s3)

        def fire_g(j, u):
            pltpu.async_copy(table_sh.at[sidx.at[j]], bufs[u], gsems[u])

        def wait_g(j, u):
            pltpu.make_async_copy(table_sh.at[sidx.at[j]], bufs[u],
                                  gsems[u]).wait()

        def fire_s(j, u):
            pltpu.async_copy(bufs[u], acc_sh.at[didx.at[j]], ssems[u],
                             add=True)

        def wait_s(j, u):
            pltpu.make_async_copy(bufs[u], acc_sh.at[didx.at[j]],
                                  ssems[u]).wait()

        for u in range(NBUF):
            fire_g(u, u)

        def block(p, carry):
            j0 = NBUF * p
            for u in range(NBUF):
                wait_g(j0 + u, u)
                fire_s(j0 + u, u)
            for u in range(NBUF):
                wait_s(j0 + u, u)
                fire_g(j0 + NBUF + u, u)
            return carry

        lax.fori_loop(0, CPT32 // NBUF - 1, block, None)
        j0f = CPT32 - NBUF
        for u in range(NBUF):
            wait_g(j0f + u, u)
            fire_s(j0f + u, u)
        for u in range(NBUF):
            wait_s(j0f + u, u)

        plsc.subcore_barrier()
        pltpu.sync_copy(acc_sh.at[pl.ds(s * RPT, RPT)], stage)
        pltpu.sync_copy(stage, out_hbm.at[c, pl.ds(s * RPT, RPT)])

    return pl.kernel(
        body,
        out_type=jax.ShapeDtypeStruct((2, NROWS, D_HID), jnp.float32),
        mesh=_mesh,
        compiler_params=pltpu.CompilerParams(needs_layout_passes=False,
                                             use_tc_tiling_on_sc=False),
        scratch_types=[
            pltpu.VMEM((CPT32, CHUNK), jnp.int32),
            pltpu.VMEM((CPT32, CHUNK), jnp.int32),
            pltpu.VMEM((CHUNK, D_HID), jnp.float32),
            pltpu.VMEM((CHUNK, D_HID), jnp.float32),
            pltpu.VMEM((CHUNK, D_HID), jnp.float32),
            pltpu.VMEM((CHUNK, D_HID), jnp.float32),
            pltpu.VMEM((RPT, D_HID), jnp.float32),
            pltpu.VMEM((RPT,), jnp.float32),
            pltpu.VMEM_SHARED((NROWS, D_HID), jnp.float32),
            pltpu.VMEM_SHARED((NROWS, D_HID), jnp.float32),
            pltpu.SemaphoreType.DMA,
            pltpu.SemaphoreType.DMA,
            pltpu.SemaphoreType.DMA,
            pltpu.SemaphoreType.DMA,
            pltpu.SemaphoreType.DMA,
            pltpu.SemaphoreType.DMA,
            pltpu.SemaphoreType.DMA,
            pltpu.SemaphoreType.DMA,
        ],
    )


_agg_scaled_call = _make_agg(True)
_agg_plain_call = _make_agg(False)


def _k2_body(x_ref, w_ref, o_ref):
    o_ref[...] = jnp.dot(x_ref[...], w_ref[...],
                         preferred_element_type=jnp.float32)


def _k4_body(parts_ref, h_ref, dis_ref, b_ref, o_ref):
    agg = parts_ref[0] + parts_ref[1] + h_ref[...] * dis_ref[...]
    z = jnp.maximum(agg * dis_ref[...] + b_ref[...], 0.0)
    o_ref[...] = z * dis_ref[...]


def _k6_body(parts_ref, z_ref, dis_ref, w_ref, b_ref, o_ref):
    u = (parts_ref[0] + parts_ref[1] + z_ref[...]) * dis_ref[...]
    o = jnp.dot(u, w_ref[...], preferred_element_type=jnp.float32) + b_ref[...]
    m = jnp.max(o, axis=1, keepdims=True)
    e = o - m
    lse = jnp.log(jnp.sum(jnp.exp(e), axis=1, keepdims=True))
    o_ref[...] = e - lse


_k2_call = pl.pallas_call(
    _k2_body, out_shape=jax.ShapeDtypeStruct((NROWS, D_HID), jnp.float32))
_k4_call = pl.pallas_call(
    _k4_body, out_shape=jax.ShapeDtypeStruct((NROWS, D_HID), jnp.float32))
_k6_call = pl.pallas_call(
    _k6_body, out_shape=jax.ShapeDtypeStruct((NROWS, D_OUT), jnp.float32))


@jax.jit
def kernel(x, edge_index, W1, b1, W2, b2):
    f32 = jnp.float32
    src = edge_index[0]
    dst = edge_index[1]
    pad = jnp.full((E_PAD - E,), PAD_IDX, dtype=jnp.int32)
    srcp = jnp.concatenate([src, pad]).reshape(EROWS, CHUNK)
    dstp = jnp.concatenate([dst, pad]).reshape(EROWS, CHUNK)
    xp = jnp.concatenate([x, jnp.zeros((NROWS - N, D_IN), f32)], axis=0)

    h1 = _k2_call(xp, W1)
    dis = _deg_call(dstp)
    dis2 = dis.reshape(NROWS, 1)
    parts1 = _agg_scaled_call(h1, dis, srcp, dstp)
    zp = _k4_call(parts1, h1, dis2, b1.reshape(1, D_HID))
    parts2 = _agg_plain_call(zp, dis, srcp, dstp)
    out = _k6_call(parts2, zp, dis2, W2, b2.reshape(1, D_OUT))
    return out[:N]


# speedup vs baseline: 65.7351x; 1.1509x over previous
import jax
import jax.numpy as jnp
from jax import lax
from jax.experimental import pallas as pl
from jax.experimental.pallas import tpu as pltpu
from jax.experimental.pallas import tpu_sc as plsc

N = 10000
E = 320000
D_IN = 128
D_HID = 16
D_OUT = 40

NROWS = 10240
CHUNK = 125
CPT32 = 80
CPT16 = 2 * CPT32
EROWS = 32 * CPT32
RPT = NROWS // 16

_mesh = plsc.VectorSubcoreMesh(core_axis_name="c", subcore_axis_name="s")


def _deg_body(edge_hbm, dis_hbm, idx_v, acc_v, part_sh, col_v, out_v):
    c = lax.axis_index("c")
    s = lax.axis_index("s")

    @pl.when(c == 0)
    def _():
        pltpu.sync_copy(edge_hbm.at[1, pl.ds(s * CPT16, CPT16)], idx_v)

        zeros16 = jnp.zeros((16,), jnp.float32)

        def zloop(i, carry):
            acc_v[pl.ds(i * 16, 16)] = zeros16
            return carry

        lax.fori_loop(0, NROWS // 16, zloop, None)

        ones16 = jnp.full((16,), 1.0, jnp.float32)

        tailmask = lax.iota(jnp.int32, 16) >= 3

        def cloop(r, carry):
            for k in range(7):
                idx = idx_v[r, pl.ds(k * 16, 16)]
                plsc.addupdate_scatter(acc_v, [idx], ones16)
            idx = idx_v[r, pl.ds(109, 16)]
            plsc.addupdate_scatter(acc_v, [idx], ones16, mask=tailmask)
            return carry

        lax.fori_loop(0, CPT16, cloop, None)

        pltpu.sync_copy(acc_v, part_sh.at[s])
        plsc.subcore_barrier()
        for r in range(16):
            pltpu.sync_copy(part_sh.at[r, pl.ds(s * RPT, RPT)], col_v.at[r])

        def sloop(i, carry):
            deg = jnp.full((16,), 1.0, jnp.float32)
            for r in range(16):
                deg = deg + col_v[r, pl.ds(i * 16, 16)]
            bi = plsc.bitcast(deg, jnp.int32)
            bi = 0x5F3759DF - lax.shift_right_arithmetic(bi, 1)
            y = plsc.bitcast(bi, jnp.float32)
            for _n in range(3):
                y = y * (1.5 - 0.5 * deg * y * y)
            out_v[pl.ds(i * 16, 16)] = y
            return carry

        lax.fori_loop(0, RPT // 16, sloop, None)
        pltpu.sync_copy(out_v, dis_hbm.at[pl.ds(s * RPT, RPT)])


_deg_call = pl.kernel(
    _deg_body,
    out_type=jax.ShapeDtypeStruct((NROWS,), jnp.float32),
    mesh=_mesh,
    compiler_params=pltpu.CompilerParams(needs_layout_passes=False, use_tc_tiling_on_sc=False),
    scratch_types=[
        pltpu.VMEM((CPT16, CHUNK), jnp.int32),
        pltpu.VMEM((NROWS,), jnp.float32),
        pltpu.VMEM_SHARED((16, NROWS), jnp.float32),
        pltpu.VMEM((16, RPT), jnp.float32),
        pltpu.VMEM((RPT,), jnp.float32),
    ],
)


NBUF = 4


def _make_agg(scale):
    def body(h_hbm, dis_hbm, edge_hbm, out_hbm,
             sidx, didx, b0, b1, b2, b3, stage, disv, table_sh, acc_sh,
             g0, g1, g2, g3, s0, s1, s2, s3):
        c = lax.axis_index("c")
        s = lax.axis_index("s")
        w = c * 16 + s
        pltpu.sync_copy(edge_hbm.at[0, pl.ds(w * CPT32, CPT32)], sidx)
        pltpu.sync_copy(edge_hbm.at[1, pl.ds(w * CPT32, CPT32)], didx)

        pltpu.sync_copy(h_hbm.at[pl.ds(s * RPT, RPT)], stage)
        if scale:
            pltpu.sync_copy(dis_hbm.at[pl.ds(s * RPT, RPT)], disv)

            def scl(i, carry):
                d = plsc.load_gather(disv, [jnp.full((16,), i, jnp.int32)])
                stage[i] = stage[i] * d
                return carry

            lax.fori_loop(0, RPT, scl, None)
        pltpu.sync_copy(stage, table_sh.at[pl.ds(s * RPT, RPT)])

        zeros16 = jnp.zeros((16,), jnp.float32)

        def zloop(i, carry):
            stage[i] = zeros16
            return carry

        lax.fori_loop(0, RPT, zloop, None)
        pltpu.sync_copy(stage, acc_sh.at[pl.ds(s * RPT, RPT)])
        plsc.subcore_barrier()

        bufs = (b0, b1, b2, b3)
        gsems = (g0, g1, g2, g3)
        ssems = (s0, s1, s2, s3)

        def fire_g(j, u):
            pltpu.async_copy(table_sh.at[sidx.at[j]], bufs[u], gsems[u])

        def wait_g(j, u):
            pltpu.make_async_copy(table_sh.at[sidx.at[j]], bufs[u],
                                  gsems[u]).wait()

        def fire_s(j, u):
            pltpu.async_copy(bufs[u], acc_sh.at[didx.at[j]], ssems[u],
                             add=True)

        def wait_s(j, u):
            pltpu.make_async_copy(bufs[u], acc_sh.at[didx.at[j]],
                                  ssems[u]).wait()

        for u in range(NBUF):
            fire_g(u, u)

        def block(p, carry):
            j0 = NBUF * p
            for u in range(NBUF):
                wait_g(j0 + u, u)
                fire_s(j0 + u, u)
            for u in range(NBUF):
                wait_s(j0 + u, u)
                fire_g(j0 + NBUF + u, u)
            return carry

        lax.fori_loop(0, CPT32 // NBUF - 1, block, None)
        j0f = CPT32 - NBUF
        for u in range(NBUF):
            wait_g(j0f + u, u)
            fire_s(j0f + u, u)
        for u in range(NBUF):
            wait_s(j0f + u, u)

        plsc.subcore_barrier()
        pltpu.sync_copy(acc_sh.at[pl.ds(s * RPT, RPT)], stage)
        pltpu.sync_copy(stage, out_hbm.at[c, pl.ds(s * RPT, RPT)])

    return pl.kernel(
        body,
        out_type=jax.ShapeDtypeStruct((2, NROWS, D_HID), jnp.float32),
        mesh=_mesh,
        compiler_params=pltpu.CompilerParams(needs_layout_passes=False,
                                             use_tc_tiling_on_sc=False),
        scratch_types=[
            pltpu.VMEM((CPT32, CHUNK), jnp.int32),
            pltpu.VMEM((CPT32, CHUNK), jnp.int32),
            pltpu.VMEM((CHUNK, D_HID), jnp.float32),
            pltpu.VMEM((CHUNK, D_HID), jnp.float32),
            pltpu.VMEM((CHUNK, D_HID), jnp.float32),
            pltpu.VMEM((CHUNK, D_HID), jnp.float32),
            pltpu.VMEM((RPT, D_HID), jnp.float32),
            pltpu.VMEM((RPT,), jnp.float32),
            pltpu.VMEM_SHARED((NROWS, D_HID), jnp.float32),
            pltpu.VMEM_SHARED((NROWS, D_HID), jnp.float32),
            pltpu.SemaphoreType.DMA,
            pltpu.SemaphoreType.DMA,
            pltpu.SemaphoreType.DMA,
            pltpu.SemaphoreType.DMA,
            pltpu.SemaphoreType.DMA,
            pltpu.SemaphoreType.DMA,
            pltpu.SemaphoreType.DMA,
            pltpu.SemaphoreType.DMA,
        ],
    )


_agg_scaled_call = _make_agg(True)
_agg_plain_call = _make_agg(False)


def _k2_body(x_ref, w_ref, o_ref):
    o_ref[pl.ds(0, N), :] = jnp.dot(x_ref[...], w_ref[...],
                                    preferred_element_type=jnp.float32)
    o_ref[pl.ds(N, NROWS - N), :] = jnp.zeros((NROWS - N, D_HID), jnp.float32)


def _k4_body(parts_ref, h_ref, dis_ref, b_ref, o_ref):
    agg = parts_ref[0] + parts_ref[1] + h_ref[...] * dis_ref[...]
    z = jnp.maximum(agg * dis_ref[...] + b_ref[...], 0.0)
    o_ref[...] = z * dis_ref[...]


def _k6_body(parts_ref, z_ref, dis_ref, w_ref, b_ref, o_ref):
    u = (parts_ref[0] + parts_ref[1] + z_ref[...]) * dis_ref[...]
    o = jnp.dot(u, w_ref[...], preferred_element_type=jnp.float32) + b_ref[...]
    m = jnp.max(o, axis=1, keepdims=True)
    e = o - m
    lse = jnp.log(jnp.sum(jnp.exp(e), axis=1, keepdims=True))
    o_ref[...] = e - lse


_k2_call = pl.pallas_call(
    _k2_body, out_shape=jax.ShapeDtypeStruct((NROWS, D_HID), jnp.float32))

_K4R = 2560
_k4_call = pl.pallas_call(
    _k4_body,
    grid=(NROWS // _K4R,),
    in_specs=[
        pl.BlockSpec((2, _K4R, D_HID), lambda i: (0, i, 0)),
        pl.BlockSpec((_K4R, D_HID), lambda i: (i, 0)),
        pl.BlockSpec((_K4R, 1), lambda i: (i, 0)),
        pl.BlockSpec((1, D_HID), lambda i: (0, 0)),
    ],
    out_specs=pl.BlockSpec((_K4R, D_HID), lambda i: (i, 0)),
    out_shape=jax.ShapeDtypeStruct((NROWS, D_HID), jnp.float32))

_K6R = 2000
_k6_call = pl.pallas_call(
    _k6_body,
    grid=(N // _K6R,),
    in_specs=[
        pl.BlockSpec((2, _K6R, D_HID), lambda i: (0, i, 0)),
        pl.BlockSpec((_K6R, D_HID), lambda i: (i, 0)),
        pl.BlockSpec((_K6R, 1), lambda i: (i, 0)),
        pl.BlockSpec((D_HID, D_OUT), lambda i: (0, 0)),
        pl.BlockSpec((1, D_OUT), lambda i: (0, 0)),
    ],
    out_specs=pl.BlockSpec((_K6R, D_OUT), lambda i: (i, 0)),
    out_shape=jax.ShapeDtypeStruct((N, D_OUT), jnp.float32))


@jax.jit
def kernel(x, edge_index, W1, b1, W2, b2):
    edge3 = edge_index.reshape(2, EROWS, CHUNK)
    h1 = _k2_call(x, W1)
    dis = _deg_call(edge3)
    dis2 = dis.reshape(NROWS, 1)
    parts1 = _agg_scaled_call(h1, dis, edge3)
    zp = _k4_call(parts1, h1, dis2, b1.reshape(1, D_HID))
    parts2 = _agg_plain_call(zp, dis, edge3)
    out = _k6_call(parts2, zp, dis2, W2, b2.reshape(1, D_OUT))
    return out


# speedup vs baseline: 71.6357x; 1.0898x over previous
import jax
import jax.numpy as jnp
from jax import lax
from jax.experimental import pallas as pl
from jax.experimental.pallas import tpu as pltpu
from jax.experimental.pallas import tpu_sc as plsc

N = 10000
E = 320000
D_IN = 128
D_HID = 16
D_OUT = 40

NROWS = 10240
CHUNK = 1250
CPT32 = 8
CPT16 = 2 * CPT32
EROWS = 32 * CPT32
RPT = NROWS // 16

_mesh = plsc.VectorSubcoreMesh(core_axis_name="c", subcore_axis_name="s")


def _deg_body(edge_hbm, dis_hbm, idx_v, acc_v, part_sh, col_v, out_v):
    c = lax.axis_index("c")
    s = lax.axis_index("s")

    @pl.when(c == 0)
    def _():
        pltpu.sync_copy(edge_hbm.at[1, pl.ds(s * CPT16, CPT16)], idx_v)

        zeros16 = jnp.zeros((16,), jnp.float32)

        def zloop(i, carry):
            acc_v[pl.ds(i * 16, 16)] = zeros16
            return carry

        lax.fori_loop(0, NROWS // 16, zloop, None)

        ones16 = jnp.full((16,), 1.0, jnp.float32)

        tailmask = lax.iota(jnp.int32, 16) >= 14

        def cloop(r, carry):
            for k in range(78):
                idx = idx_v[r, pl.ds(k * 16, 16)]
                plsc.addupdate_scatter(acc_v, [idx], ones16)
            idx = idx_v[r, pl.ds(1234, 16)]
            plsc.addupdate_scatter(acc_v, [idx], ones16, mask=tailmask)
            return carry

        lax.fori_loop(0, CPT16, cloop, None)

        pltpu.sync_copy(acc_v, part_sh.at[s])
        plsc.subcore_barrier()
        for r in range(16):
            pltpu.sync_copy(part_sh.at[r, pl.ds(s * RPT, RPT)], col_v.at[r])

        def sloop(i, carry):
            deg = jnp.full((16,), 1.0, jnp.float32)
            for r in range(16):
                deg = deg + col_v[r, pl.ds(i * 16, 16)]
            bi = plsc.bitcast(deg, jnp.int32)
            bi = 0x5F3759DF - lax.shift_right_arithmetic(bi, 1)
            y = plsc.bitcast(bi, jnp.float32)
            for _n in range(3):
                y = y * (1.5 - 0.5 * deg * y * y)
            out_v[pl.ds(i * 16, 16)] = y
            return carry

        lax.fori_loop(0, RPT // 16, sloop, None)
        pltpu.sync_copy(out_v, dis_hbm.at[pl.ds(s * RPT, RPT)])


_deg_call = pl.kernel(
    _deg_body,
    out_type=jax.ShapeDtypeStruct((NROWS,), jnp.float32),
    mesh=_mesh,
    compiler_params=pltpu.CompilerParams(needs_layout_passes=False, use_tc_tiling_on_sc=False),
    scratch_types=[
        pltpu.VMEM((CPT16, CHUNK), jnp.int32),
        pltpu.VMEM((NROWS,), jnp.float32),
        pltpu.VMEM_SHARED((16, NROWS), jnp.float32),
        pltpu.VMEM((16, RPT), jnp.float32),
        pltpu.VMEM((RPT,), jnp.float32),
    ],
)


NBUF = 4


def _make_agg(scale):
    def body(h_hbm, dis_hbm, edge_hbm, zero_hbm, out_hbm,
             sidx, didx, b0, b1, stage, disv, table_sh, acc_sh,
             g0, g1, s0, s1):
        c = lax.axis_index("c")
        s = lax.axis_index("s")
        w = c * 16 + s
        pltpu.sync_copy(edge_hbm.at[0, pl.ds(w * CPT32, CPT32)], sidx)
        pltpu.sync_copy(edge_hbm.at[1, pl.ds(w * CPT32, CPT32)], didx)

        pltpu.sync_copy(h_hbm.at[pl.ds(s * RPT, RPT)], stage)
        if scale:
            pltpu.sync_copy(dis_hbm.at[pl.ds(s * RPT, RPT)], disv)

            def scl(i, carry):
                d = plsc.load_gather(disv, [jnp.full((16,), i, jnp.int32)])
                stage[i] = stage[i] * d
                return carry

            lax.fori_loop(0, RPT, scl, None)
        pltpu.sync_copy(stage, table_sh.at[pl.ds(s * RPT, RPT)])
        pltpu.sync_copy(zero_hbm.at[pl.ds(0, RPT)], stage)
        pltpu.sync_copy(stage, acc_sh.at[pl.ds(s * RPT, RPT)])
        plsc.subcore_barrier()

        bufs = (b0, b1)
        gsems = (g0, g1)
        ssems = (s0, s1)

        def idx_g(m):
            return sidx.at[m]

        def idx_s(m):
            return didx.at[m]

        def fire_g(m, u):
            pltpu.async_copy(table_sh.at[idx_g(m)], bufs[u], gsems[u])

        def wait_g(m, u):
            pltpu.make_async_copy(table_sh.at[idx_g(m)], bufs[u],
                                  gsems[u]).wait()

        def fire_s(m, u):
            pltpu.async_copy(bufs[u], acc_sh.at[idx_s(m)], ssems[u],
                             add=True)

        def wait_s(m, u):
            pltpu.make_async_copy(bufs[u], acc_sh.at[idx_s(m)],
                                  ssems[u]).wait()

        fire_g(0, 0)
        fire_g(1, 1)
        for m in range(CPT32):
            u = m & 1
            wait_g(m, u)
            fire_s(m, u)
            if m + 2 < CPT32:
                wait_s(m, u)
                fire_g(m + 2, u)
        wait_s(CPT32 - 2, (CPT32 - 2) & 1)
        wait_s(CPT32 - 1, (CPT32 - 1) & 1)

        plsc.subcore_barrier()
        pltpu.sync_copy(acc_sh.at[pl.ds(s * RPT, RPT)], stage)
        pltpu.sync_copy(stage, out_hbm.at[c, pl.ds(s * RPT, RPT)])

    return pl.kernel(
        body,
        out_type=jax.ShapeDtypeStruct((2, NROWS, D_HID), jnp.float32),
        mesh=_mesh,
        compiler_params=pltpu.CompilerParams(needs_layout_passes=False,
                                             use_tc_tiling_on_sc=False),
        scratch_types=[
            pltpu.VMEM((CPT32, CHUNK), jnp.int32),
            pltpu.VMEM((CPT32, CHUNK), jnp.int32),
            pltpu.VMEM((CHUNK, D_HID), jnp.float32),
            pltpu.VMEM((CHUNK, D_HID), jnp.float32),
            pltpu.VMEM((RPT, D_HID), jnp.float32),
            pltpu.VMEM((RPT,), jnp.float32),
            pltpu.VMEM_SHARED((NROWS, D_HID), jnp.float32),
            pltpu.VMEM_SHARED((NROWS, D_HID), jnp.float32),
            pltpu.SemaphoreType.DMA,
            pltpu.SemaphoreType.DMA,
            pltpu.SemaphoreType.DMA,
            pltpu.SemaphoreType.DMA,
        ],
    )


_agg_scaled_call = _make_agg(True)
_agg_plain_call = _make_agg(False)


def _k2_body(x_ref, w_ref, o_ref):
    o_ref[pl.ds(0, N), :] = jnp.dot(x_ref[...], w_ref[...],
                                    preferred_element_type=jnp.float32)
    o_ref[pl.ds(N, NROWS - N), :] = jnp.zeros((NROWS - N, D_HID), jnp.float32)


def _k4_body(parts_ref, h_ref, dis_ref, b_ref, o_ref):
    agg = parts_ref[0] + parts_ref[1] + h_ref[...] * dis_ref[...]
    z = jnp.maximum(agg * dis_ref[...] + b_ref[...], 0.0)
    o_ref[...] = z * dis_ref[...]


def _k6_body(parts_ref, z_ref, dis_ref, w_ref, b_ref, o_ref):
    u = (parts_ref[0] + parts_ref[1] + z_ref[...]) * dis_ref[...]
    o = jnp.dot(u, w_ref[...], preferred_element_type=jnp.float32) + b_ref[...]
    m = jnp.max(o, axis=1, keepdims=True)
    e = o - m
    lse = jnp.log(jnp.sum(jnp.exp(e), axis=1, keepdims=True))
    o_ref[...] = e - lse


_k2_call = pl.pallas_call(
    _k2_body, out_shape=jax.ShapeDtypeStruct((NROWS, D_HID), jnp.float32))

_K4R = 2560
_k4_call = pl.pallas_call(
    _k4_body,
    grid=(NROWS // _K4R,),
    in_specs=[
        pl.BlockSpec((2, _K4R, D_HID), lambda i: (0, i, 0)),
        pl.BlockSpec((_K4R, D_HID), lambda i: (i, 0)),
        pl.BlockSpec((_K4R, 1), lambda i: (i, 0)),
        pl.BlockSpec((1, D_HID), lambda i: (0, 0)),
    ],
    out_specs=pl.BlockSpec((_K4R, D_HID), lambda i: (i, 0)),
    out_shape=jax.ShapeDtypeStruct((NROWS, D_HID), jnp.float32))

_K6R = 2000
_k6_call = pl.pallas_call(
    _k6_body,
    grid=(N // _K6R,),
    in_specs=[
        pl.BlockSpec((2, _K6R, D_HID), lambda i: (0, i, 0)),
        pl.BlockSpec((_K6R, D_HID), lambda i: (i, 0)),
        pl.BlockSpec((_K6R, 1), lambda i: (i, 0)),
        pl.BlockSpec((D_HID, D_OUT), lambda i: (0, 0)),
        pl.BlockSpec((1, D_OUT), lambda i: (0, 0)),
    ],
    out_specs=pl.BlockSpec((_K6R, D_OUT), lambda i: (i, 0)),
    out_shape=jax.ShapeDtypeStruct((N, D_OUT), jnp.float32))


@jax.jit
def kernel(x, edge_index, W1, b1, W2, b2):
    edge3 = edge_index.reshape(2, EROWS, CHUNK)
    h1 = _k2_call(x, W1)
    dis = _deg_call(edge3)
    dis2 = dis.reshape(NROWS, 1)
    zrows = jnp.zeros((RPT, D_HID), jnp.float32)
    parts1 = _agg_scaled_call(h1, dis, edge3, zrows)
    zp = _k4_call(parts1, h1, dis2, b1.reshape(1, D_HID))
    parts2 = _agg_plain_call(zp, dis, edge3, zrows)
    out = _k6_call(parts2, zp, dis2, W2, b2.reshape(1, D_OUT))
    return out


# speedup vs baseline: 86.7788x; 1.2114x over previous
import jax
import jax.numpy as jnp
from jax import lax
from jax.experimental import pallas as pl
from jax.experimental.pallas import tpu as pltpu
from jax.experimental.pallas import tpu_sc as plsc

N = 10000
E = 320000
D_IN = 128
D_HID = 16
D_OUT = 40

NROWS = 10240
CHUNK = 1250
CPT32 = 8
CPT16 = 2 * CPT32
EROWS = 32 * CPT32
RPT = NROWS // 16

_mesh = plsc.VectorSubcoreMesh(core_axis_name="c", subcore_axis_name="s")


def _deg_body(edge_hbm, dis_hbm, idx_v, acc_v, part_sh, col_v, out_v, ybuf):
    c = lax.axis_index("c")
    s = lax.axis_index("s")

    @pl.when(c == 0)
    def _():
        pltpu.sync_copy(edge_hbm.at[1, pl.ds(s * CPT16, CPT16)], idx_v)

        zeros16 = jnp.zeros((16,), jnp.float32)

        def zloop(i, carry):
            acc_v[pl.ds(i * 16, 16)] = zeros16
            return carry

        lax.fori_loop(0, NROWS // 16, zloop, None)

        ones16 = jnp.full((16,), 1.0, jnp.float32)

        tailmask = lax.iota(jnp.int32, 16) >= 14

        def cloop(r, carry):
            for k in range(78):
                idx = idx_v[r, pl.ds(k * 16, 16)]
                plsc.addupdate_scatter(acc_v, [idx], ones16)
            idx = idx_v[r, pl.ds(1234, 16)]
            plsc.addupdate_scatter(acc_v, [idx], ones16, mask=tailmask)
            return carry

        lax.fori_loop(0, CPT16, cloop, None)

        pltpu.sync_copy(acc_v, part_sh.at[s])
        plsc.subcore_barrier()
        for r in range(16):
            pltpu.sync_copy(part_sh.at[r, pl.ds(s * RPT, RPT)], col_v.at[r])

        def sloop(i, carry):
            deg = jnp.full((16,), 1.0, jnp.float32)
            for r in range(16):
                deg = deg + col_v[r, pl.ds(i * 16, 16)]
            bi = plsc.bitcast(deg, jnp.int32)
            bi = 0x5F3759DF - lax.shift_right_arithmetic(bi, 1)
            y = plsc.bitcast(bi, jnp.float32)
            for _n in range(3):
                y = y * (1.5 - 0.5 * deg * y * y)
            ybuf[...] = y
            for u in range(16):
                d = plsc.load_gather(ybuf, [jnp.full((16,), u, jnp.int32)])
                out_v[2 * i + (u // 8), pl.ds((u % 8) * 16, 16)] = d
            return carry

        lax.fori_loop(0, RPT // 16, sloop, None)
        pltpu.sync_copy(out_v, dis_hbm.at[pl.ds(s * (RPT // 8), RPT // 8)])


_deg_call = pl.kernel(
    _deg_body,
    out_type=jax.ShapeDtypeStruct((NROWS // 8, 128), jnp.float32),
    mesh=_mesh,
    compiler_params=pltpu.CompilerParams(needs_layout_passes=False, use_tc_tiling_on_sc=False),
    scratch_types=[
        pltpu.VMEM((CPT16, CHUNK), jnp.int32),
        pltpu.VMEM((NROWS,), jnp.float32),
        pltpu.VMEM_SHARED((16, NROWS), jnp.float32),
        pltpu.VMEM((16, RPT), jnp.float32),
        pltpu.VMEM((RPT // 8, 128), jnp.float32),
        pltpu.VMEM((16,), jnp.float32),
    ],
)


NBUF = 4


def _make_agg(scale):
    def body(h_hbm, dis_hbm, edge_hbm, zero_hbm, out_hbm,
             sidx, didx, b0, b1, stage, disv, table_sh, acc_sh,
             g0, g1, s0, s1):
        c = lax.axis_index("c")
        s = lax.axis_index("s")
        w = c * 16 + s
        pltpu.sync_copy(edge_hbm.at[0, pl.ds(w * CPT32, CPT32)], sidx)
        pltpu.sync_copy(edge_hbm.at[1, pl.ds(w * CPT32, CPT32)], didx)

        pltpu.sync_copy(h_hbm.at[pl.ds(s * RPT, RPT)], stage)
        if scale:
            pltpu.sync_copy(dis_hbm.at[pl.ds(s * (RPT // 8), RPT // 8)], disv)

            def scl(r, carry):
                for k in range(8):
                    i = r * 8 + k
                    stage[i] = stage[i] * disv[r, pl.ds(k * 16, 16)]
                return carry

            lax.fori_loop(0, RPT // 8, scl, None)
        pltpu.sync_copy(stage, table_sh.at[pl.ds(s * RPT, RPT)])

        @pl.when(c != 0)
        def _():
            pltpu.sync_copy(zero_hbm, stage)

        pltpu.sync_copy(stage, acc_sh.at[pl.ds(s * RPT, RPT)])
        plsc.subcore_barrier()

        bufs = (b0, b1)
        gsems = (g0, g1)
        ssems = (s0, s1)

        def idx_g(m):
            return sidx.at[m]

        def idx_s(m):
            return didx.at[m]

        def fire_g(m, u):
            pltpu.async_copy(table_sh.at[idx_g(m)], bufs[u], gsems[u])

        def wait_g(m, u):
            pltpu.make_async_copy(table_sh.at[idx_g(m)], bufs[u],
                                  gsems[u]).wait()

        def fire_s(m, u):
            pltpu.async_copy(bufs[u], acc_sh.at[idx_s(m)], ssems[u],
                             add=True)

        def wait_s(m, u):
            pltpu.make_async_copy(bufs[u], acc_sh.at[idx_s(m)],
                                  ssems[u]).wait()

        fire_g(0, 0)
        fire_g(1, 1)
        for m in range(CPT32):
            u = m & 1
            wait_g(m, u)
            fire_s(m, u)
            if m + 2 < CPT32:
                wait_s(m, u)
                fire_g(m + 2, u)
        wait_s(CPT32 - 2, (CPT32 - 2) & 1)
        wait_s(CPT32 - 1, (CPT32 - 1) & 1)

        plsc.subcore_barrier()
        pltpu.sync_copy(acc_sh.at[pl.ds(s * RPT, RPT)], stage)
        pltpu.sync_copy(stage, out_hbm.at[c, pl.ds(s * RPT, RPT)])

    return pl.kernel(
        body,
        out_type=jax.ShapeDtypeStruct((2, NROWS, D_HID), jnp.float32),
        mesh=_mesh,
        compiler_params=pltpu.CompilerParams(needs_layout_passes=False,
                                             use_tc_tiling_on_sc=False),
        scratch_types=[
            pltpu.VMEM((CPT32, CHUNK), jnp.int32),
            pltpu.VMEM((CPT32, CHUNK), jnp.int32),
            pltpu.VMEM((CHUNK, D_HID), jnp.float32),
            pltpu.VMEM((CHUNK, D_HID), jnp.float32),
            pltpu.VMEM((RPT, D_HID), jnp.float32),
            pltpu.VMEM((RPT // 8, 128), jnp.float32),
            pltpu.VMEM_SHARED((NROWS, D_HID), jnp.float32),
            pltpu.VMEM_SHARED((NROWS, D_HID), jnp.float32),
            pltpu.SemaphoreType.DMA,
            pltpu.SemaphoreType.DMA,
            pltpu.SemaphoreType.DMA,
            pltpu.SemaphoreType.DMA,
        ],
    )


_agg_scaled_call = _make_agg(True)
_agg_plain_call = _make_agg(False)


def _k2_body(x_ref, w_ref, o_ref):
    o_ref[pl.ds(0, N), :] = jnp.dot(x_ref[...], w_ref[...],
                                    preferred_element_type=jnp.float32,
                                    precision=lax.Precision.HIGHEST)
    o_ref[pl.ds(N, NROWS - N), :] = jnp.zeros((NROWS - N, D_HID), jnp.float32)


def _k4_body(parts_ref, dis_ref, b_ref, o_ref):
    agg = parts_ref[0] + parts_ref[1]
    z = jnp.maximum(agg * dis_ref[...] + b_ref[...], 0.0)
    o_ref[...] = z * dis_ref[...]


def _k6_body(parts_ref, dis_ref, w_ref, b_ref, o_ref):
    u = (parts_ref[0] + parts_ref[1]) * dis_ref[...]
    o = jnp.dot(u, w_ref[...], preferred_element_type=jnp.float32,
                precision=lax.Precision.HIGHEST) + b_ref[...]
    o3 = o.reshape(o.shape[0], 8, D_OUT)
    m = jnp.max(o3, axis=2, keepdims=True)
    e = o3 - m
    lse = jnp.log(jnp.sum(jnp.exp(e), axis=2, keepdims=True))
    o_ref[...] = (e - lse).reshape(o.shape[0], 8 * D_OUT)


NRV = NROWS // 8

_k2_call = pl.pallas_call(
    _k2_body, out_shape=jax.ShapeDtypeStruct((NROWS, D_HID), jnp.float32))

_K4R = 320
_k4_call = pl.pallas_call(
    _k4_body,
    grid=(NRV // _K4R,),
    in_specs=[
        pl.BlockSpec((2, _K4R, 128), lambda i: (0, i, 0)),
        pl.BlockSpec((_K4R, 128), lambda i: (i, 0)),
        pl.BlockSpec((1, 128), lambda i: (0, 0)),
    ],
    out_specs=pl.BlockSpec((_K4R, 128), lambda i: (i, 0)),
    out_shape=jax.ShapeDtypeStruct((NRV, 128), jnp.float32))

_K6R = 320
_k6_call = pl.pallas_call(
    _k6_body,
    grid=(NRV // _K6R,),
    in_specs=[
        pl.BlockSpec((2, _K6R, 128), lambda i: (0, i, 0)),
        pl.BlockSpec((_K6R, 128), lambda i: (i, 0)),
        pl.BlockSpec((128, 8 * D_OUT), lambda i: (0, 0)),
        pl.BlockSpec((1, 8 * D_OUT), lambda i: (0, 0)),
    ],
    out_specs=pl.BlockSpec((_K6R, 8 * D_OUT), lambda i: (i, 0)),
    out_shape=jax.ShapeDtypeStruct((NRV, 8 * D_OUT), jnp.float32))


@jax.jit
def kernel(x, edge_index, W1, b1, W2, b2):
    edge3 = edge_index.reshape(2, EROWS, CHUNK)
    h1 = _k2_call(x, W1)
    dis16 = _deg_call(edge3)
    zrows = jnp.zeros((RPT, D_HID), jnp.float32)
    parts1 = _agg_scaled_call(h1, dis16, edge3, zrows)
    b1t = jnp.tile(b1, (8,)).reshape(1, 128)
    zp = _k4_call(parts1.reshape(2, NRV, 128), dis16, b1t)
    parts2 = _agg_plain_call(zp.reshape(NROWS, D_HID), dis16, edge3, zrows)
    w2rep = jnp.kron(jnp.eye(8, dtype=jnp.float32), W2)
    b2t = jnp.tile(b2, (8,)).reshape(1, 8 * D_OUT)
    out320 = _k6_call(parts2.reshape(2, NRV, 128), dis16, w2rep, b2t)
    return out320.reshape(NROWS, D_OUT)[:N]


# speedup vs baseline: 88.0447x; 1.0146x over previous
import jax
import jax.numpy as jnp
from jax import lax
from jax.experimental import pallas as pl
from jax.experimental.pallas import tpu as pltpu
from jax.experimental.pallas import tpu_sc as plsc

N = 10000
E = 320000
D_IN = 128
D_HID = 16
D_OUT = 40

NROWS = 10240
CHUNK = 1250
CPT32 = 8
CPT16 = 2 * CPT32
EROWS = 32 * CPT32
RPT = NROWS // 16

_mesh = plsc.VectorSubcoreMesh(core_axis_name="c", subcore_axis_name="s")


def _deg_body(edge_hbm, zero_hbm, deg_hbm, idx_v, acc_v, part_sh, col_v):
    c = lax.axis_index("c")
    s = lax.axis_index("s")
    w = c * 16 + s
    pltpu.sync_copy(edge_hbm.at[1, pl.ds(w * CPT32, CPT32)], idx_v)
    pltpu.sync_copy(zero_hbm, acc_v)

    ones16 = jnp.full((16,), 1.0, jnp.float32)
    tailmask = lax.iota(jnp.int32, 16) >= 14

    def cloop(r, carry):
        for k in range(78):
            idx = idx_v[r, pl.ds(k * 16, 16)]
            plsc.addupdate_scatter(acc_v, [idx], ones16)
        idx = idx_v[r, pl.ds(1234, 16)]
        plsc.addupdate_scatter(acc_v, [idx], ones16, mask=tailmask)
        return carry

    lax.fori_loop(0, CPT32, cloop, None)

    pltpu.sync_copy(acc_v, part_sh.at[s])
    plsc.subcore_barrier()
    for r in range(16):
        pltpu.sync_copy(part_sh.at[r, pl.ds(s * RPT, RPT)], col_v.at[r])

    def sloop(i, carry):
        tot = jnp.zeros((16,), jnp.float32)
        for r in range(16):
            tot = tot + col_v[r, pl.ds(i * 16, 16)]
        acc_v[pl.ds(i * 16, 16)] = tot
        return carry

    lax.fori_loop(0, RPT // 16, sloop, None)
    pltpu.sync_copy(acc_v.at[pl.ds(0, RPT)], deg_hbm.at[c, pl.ds(s * RPT, RPT)])


_deg_call = pl.kernel(
    _deg_body,
    out_type=jax.ShapeDtypeStruct((2, NROWS), jnp.float32),
    mesh=_mesh,
    compiler_params=pltpu.CompilerParams(needs_layout_passes=False, use_tc_tiling_on_sc=False),
    scratch_types=[
        pltpu.VMEM((CPT32, CHUNK), jnp.int32),
        pltpu.VMEM((NROWS,), jnp.float32),
        pltpu.VMEM_SHARED((16, NROWS), jnp.float32),
        pltpu.VMEM((16, RPT), jnp.float32),
    ],
)


NBUF = 4


def _make_agg(scale):
    def body(h_hbm, deg_hbm, edge_hbm, zero_hbm, out_hbm, dis_hbm,
             sidx, didx, b0, b1, stage, disv, dbuf, ybuf, table_sh, acc_sh,
             g0, g1, s0, s1):
        c = lax.axis_index("c")
        s = lax.axis_index("s")
        w = c * 16 + s
        RV = RPT // 8
        pltpu.sync_copy(edge_hbm.at[0, pl.ds(w * CPT32, CPT32)], sidx)
        pltpu.sync_copy(edge_hbm.at[1, pl.ds(w * CPT32, CPT32)], didx)

        pltpu.sync_copy(h_hbm.at[pl.ds(s * RPT, RPT)], stage)
        if scale:
            pltpu.sync_copy(deg_hbm.at[0, pl.ds(s * RPT, RPT)],
                            dbuf.at[0])
            pltpu.sync_copy(deg_hbm.at[1, pl.ds(s * RPT, RPT)],
                            dbuf.at[1])

            def dloop(i, carry):
                deg = jnp.full((16,), 1.0, jnp.float32)
                deg = deg + dbuf[0, pl.ds(i * 16, 16)]
                deg = deg + dbuf[1, pl.ds(i * 16, 16)]
                bi = plsc.bitcast(deg, jnp.int32)
                bi = 0x5F3759DF - lax.shift_right_arithmetic(bi, 1)
                y = plsc.bitcast(bi, jnp.float32)
                for _n in range(3):
                    y = y * (1.5 - 0.5 * deg * y * y)
                ybuf[...] = y
                for u in range(16):
                    d = plsc.load_gather(ybuf, [jnp.full((16,), u, jnp.int32)])
                    disv[2 * i + (u // 8), pl.ds((u % 8) * 16, 16)] = d
                return carry

            lax.fori_loop(0, RPT // 16, dloop, None)

            @pl.when(c == 0)
            def _():
                pltpu.sync_copy(disv, dis_hbm.at[pl.ds(s * RV, RV)])

            def scl(r, carry):
                for k in range(8):
                    i = r * 8 + k
                    stage[i] = stage[i] * disv[r, pl.ds(k * 16, 16)]
                return carry

            lax.fori_loop(0, RV, scl, None)
        pltpu.sync_copy(stage, table_sh.at[pl.ds(s * RPT, RPT)])

        @pl.when(c != 0)
        def _():
            pltpu.sync_copy(zero_hbm, stage)

        pltpu.sync_copy(stage, acc_sh.at[pl.ds(s * RPT, RPT)])
        plsc.subcore_barrier()

        bufs = (b0, b1)
        gsems = (g0, g1)
        ssems = (s0, s1)

        def idx_g(m):
            return sidx.at[m]

        def idx_s(m):
            return didx.at[m]

        def fire_g(m, u):
            pltpu.async_copy(table_sh.at[idx_g(m)], bufs[u], gsems[u])

        def wait_g(m, u):
            pltpu.make_async_copy(table_sh.at[idx_g(m)], bufs[u],
                                  gsems[u]).wait()

        def fire_s(m, u):
            pltpu.async_copy(bufs[u], acc_sh.at[idx_s(m)], ssems[u],
                             add=True)

        def wait_s(m, u):
            pltpu.make_async_copy(bufs[u], acc_sh.at[idx_s(m)],
                                  ssems[u]).wait()

        fire_g(0, 0)
        fire_g(1, 1)
        for m in range(CPT32):
            u = m & 1
            wait_g(m, u)
            fire_s(m, u)
            if m + 2 < CPT32:
                wait_s(m, u)
                fire_g(m + 2, u)
        wait_s(CPT32 - 2, (CPT32 - 2) & 1)
        wait_s(CPT32 - 1, (CPT32 - 1) & 1)

        plsc.subcore_barrier()
        pltpu.sync_copy(acc_sh.at[pl.ds(s * RPT, RPT)], stage)
        pltpu.sync_copy(stage, out_hbm.at[c, pl.ds(s * RPT, RPT)])

    outs = (jax.ShapeDtypeStruct((2, NROWS, D_HID), jnp.float32),
            jax.ShapeDtypeStruct((NROWS // 8, 128), jnp.float32))
    if not scale:
        outs = (jax.ShapeDtypeStruct((2, NROWS, D_HID), jnp.float32),
                jax.ShapeDtypeStruct((8, 128), jnp.float32))
    return pl.kernel(
        body,
        out_type=outs,
        mesh=_mesh,
        compiler_params=pltpu.CompilerParams(needs_layout_passes=False,
                                             use_tc_tiling_on_sc=False),
        scratch_types=[
            pltpu.VMEM((CPT32, CHUNK), jnp.int32),
            pltpu.VMEM((CPT32, CHUNK), jnp.int32),
            pltpu.VMEM((CHUNK, D_HID), jnp.float32),
            pltpu.VMEM((CHUNK, D_HID), jnp.float32),
            pltpu.VMEM((RPT, D_HID), jnp.float32),
            pltpu.VMEM((RPT // 8, 128), jnp.float32),
            pltpu.VMEM((2, RPT), jnp.float32),
            pltpu.VMEM((16,), jnp.float32),
            pltpu.VMEM_SHARED((NROWS, D_HID), jnp.float32),
            pltpu.VMEM_SHARED((NROWS, D_HID), jnp.float32),
            pltpu.SemaphoreType.DMA,
            pltpu.SemaphoreType.DMA,
            pltpu.SemaphoreType.DMA,
            pltpu.SemaphoreType.DMA,
        ],
    )


_agg_scaled_call = _make_agg(True)
_agg_plain_call = _make_agg(False)


def _k2_body(x_ref, w_ref, o_ref):
    o_ref[pl.ds(0, N), :] = jnp.dot(x_ref[...], w_ref[...],
                                    preferred_element_type=jnp.float32,
                                    precision=lax.Precision.HIGHEST)
    o_ref[pl.ds(N, NROWS - N), :] = jnp.zeros((NROWS - N, D_HID), jnp.float32)


def _k4_body(parts_ref, dis_ref, b_ref, o_ref):
    agg = parts_ref[0] + parts_ref[1]
    z = jnp.maximum(agg * dis_ref[...] + b_ref[...], 0.0)
    o_ref[...] = z * dis_ref[...]


def _k6_body(parts_ref, dis_ref, w_ref, b_ref, o_ref):
    u = (parts_ref[0] + parts_ref[1]) * dis_ref[...]
    o = jnp.dot(u, w_ref[...], preferred_element_type=jnp.float32,
                precision=lax.Precision.HIGHEST) + b_ref[...]
    o3 = o.reshape(o.shape[0], 8, D_OUT)
    m = jnp.max(o3, axis=2, keepdims=True)
    e = o3 - m
    lse = jnp.log(jnp.sum(jnp.exp(e), axis=2, keepdims=True))
    o_ref[...] = (e - lse).reshape(o.shape[0], 8 * D_OUT)


NRV = NROWS // 8

_k2_call = pl.pallas_call(
    _k2_body, out_shape=jax.ShapeDtypeStruct((NROWS, D_HID), jnp.float32))

_K4R = 320
_k4_call = pl.pallas_call(
    _k4_body,
    grid=(NRV // _K4R,),
    in_specs=[
        pl.BlockSpec((2, _K4R, 128), lambda i: (0, i, 0)),
        pl.BlockSpec((_K4R, 128), lambda i: (i, 0)),
        pl.BlockSpec((1, 128), lambda i: (0, 0)),
    ],
    out_specs=pl.BlockSpec((_K4R, 128), lambda i: (i, 0)),
    out_shape=jax.ShapeDtypeStruct((NRV, 128), jnp.float32))

_K6R = 320
_k6_call = pl.pallas_call(
    _k6_body,
    grid=(NRV // _K6R,),
    in_specs=[
        pl.BlockSpec((2, _K6R, 128), lambda i: (0, i, 0)),
        pl.BlockSpec((_K6R, 128), lambda i: (i, 0)),
        pl.BlockSpec((128, 8 * D_OUT), lambda i: (0, 0)),
        pl.BlockSpec((1, 8 * D_OUT), lambda i: (0, 0)),
    ],
    out_specs=pl.BlockSpec((_K6R, 8 * D_OUT), lambda i: (i, 0)),
    out_shape=jax.ShapeDtypeStruct((NRV, 8 * D_OUT), jnp.float32))


@jax.jit
def kernel(x, edge_index, W1, b1, W2, b2):
    edge3 = edge_index.reshape(2, EROWS, CHUNK)
    h1 = _k2_call(x, W1)
    zdeg = jnp.zeros((NROWS,), jnp.float32)
    deg2 = _deg_call(edge3, zdeg)
    zrows = jnp.zeros((RPT, D_HID), jnp.float32)
    parts1, dis16 = _agg_scaled_call(h1, deg2, edge3, zrows)
    b1t = jnp.tile(b1, (8,)).reshape(1, 128)
    zp = _k4_call(parts1.reshape(2, NRV, 128), dis16, b1t)
    parts2, _unused = _agg_plain_call(zp.reshape(NROWS, D_HID), deg2, edge3,
                                      zrows)
    w2rep = jnp.kron(jnp.eye(8, dtype=jnp.float32), W2)
    b2t = jnp.tile(b2, (8,)).reshape(1, 8 * D_OUT)
    out320 = _k6_call(parts2.reshape(2, NRV, 128), dis16, w2rep, b2t)
    return out320.reshape(NROWS, D_OUT)[:N]


# speedup vs baseline: 90.9439x; 1.0329x over previous
import jax
import jax.numpy as jnp
from jax import lax
from jax.experimental import pallas as pl
from jax.experimental.pallas import tpu as pltpu
from jax.experimental.pallas import tpu_sc as plsc

N = 10000
E = 320000
D_IN = 128
D_HID = 16
D_OUT = 40

NROWS = 10240
CHUNK = 1000
CPT32 = 10
EPT = CHUNK * CPT32
RPT = NROWS // 16

_mesh = plsc.VectorSubcoreMesh(core_axis_name="c", subcore_axis_name="s")


def _deg_body(edge_hbm, zero_hbm, deg_hbm, idx_v, acc_v, part_sh, col_v):
    c = lax.axis_index("c")
    s = lax.axis_index("s")
    w = c * 16 + s
    pltpu.sync_copy(edge_hbm.at[1, pl.ds(w * EPT, EPT)], idx_v)
    pltpu.sync_copy(zero_hbm, acc_v)

    ones16 = jnp.full((16,), 1.0, jnp.float32)

    def cloop(r, carry):
        for k in range(8):
            idx = idx_v[pl.ds(r * 128 + k * 16, 16)]
            plsc.addupdate_scatter(acc_v, [idx], ones16)
        return carry

    lax.fori_loop(0, EPT // 128, cloop, None)

    pltpu.sync_copy(acc_v, part_sh.at[s])
    plsc.subcore_barrier()
    for r in range(16):
        pltpu.sync_copy(part_sh.at[r, pl.ds(s * RPT, RPT)], col_v.at[r])

    def sloop(i, carry):
        tot = jnp.zeros((16,), jnp.float32)
        for r in range(16):
            tot = tot + col_v[r, pl.ds(i * 16, 16)]
        acc_v[pl.ds(i * 16, 16)] = tot
        return carry

    lax.fori_loop(0, RPT // 16, sloop, None)
    pltpu.sync_copy(acc_v.at[pl.ds(0, RPT)], deg_hbm.at[c, pl.ds(s * RPT, RPT)])


_deg_call = pl.kernel(
    _deg_body,
    out_type=jax.ShapeDtypeStruct((2, NROWS), jnp.float32),
    mesh=_mesh,
    compiler_params=pltpu.CompilerParams(needs_layout_passes=False, use_tc_tiling_on_sc=False),
    scratch_types=[
        pltpu.VMEM((EPT,), jnp.int32),
        pltpu.VMEM((NROWS,), jnp.float32),
        pltpu.VMEM_SHARED((16, NROWS), jnp.float32),
        pltpu.VMEM((16, RPT), jnp.float32),
    ],
)


NBUF = 4


def _make_agg(scale):
    def body(h_hbm, deg_hbm, edge_hbm, zero_hbm, out_hbm, dis_hbm,
             sidx, didx, b0, b1, b2, stage, disv, dbuf, ybuf, table_sh, acc_sh,
             g0, g1, g2, s0, s1, s2):
        c = lax.axis_index("c")
        s = lax.axis_index("s")
        w = c * 16 + s
        pltpu.sync_copy(edge_hbm.at[0, pl.ds(w * EPT, EPT)], sidx)
        pltpu.sync_copy(edge_hbm.at[1, pl.ds(w * EPT, EPT)], didx)

        pltpu.sync_copy(h_hbm.at[pl.ds(s * RPT, RPT)], stage)
        if scale:
            pltpu.sync_copy(deg_hbm.at[0, pl.ds(s * RPT, RPT)],
                            dbuf.at[0])
            pltpu.sync_copy(deg_hbm.at[1, pl.ds(s * RPT, RPT)],
                            dbuf.at[1])

            def dloop(i, carry):
                deg = jnp.full((16,), 1.0, jnp.float32)
                deg = deg + dbuf[0, pl.ds(i * 16, 16)]
                deg = deg + dbuf[1, pl.ds(i * 16, 16)]
                bi = plsc.bitcast(deg, jnp.int32)
                bi = 0x5F3759DF - lax.shift_right_arithmetic(bi, 1)
                y = plsc.bitcast(bi, jnp.float32)
                for _n in range(3):
                    y = y * (1.5 - 0.5 * deg * y * y)
                ybuf[...] = y
                for u in range(16):
                    d = plsc.load_gather(ybuf, [jnp.full((16,), u, jnp.int32)])
                    disv[2 * i + (u // 8), pl.ds((u % 8) * 16, 16)] = d
                    r = i * 16 + u
                    stage[r] = stage[r] * d
                return carry

            lax.fori_loop(0, RPT // 16, dloop, None)

            @pl.when(c == 0)
            def _():
                pltpu.sync_copy(disv, dis_hbm.at[pl.ds(s * (RPT // 8),
                                                       RPT // 8)])
        pltpu.sync_copy(stage, table_sh.at[pl.ds(s * RPT, RPT)])

        @pl.when(c != 0)
        def _():
            pltpu.sync_copy(zero_hbm, stage)

        pltpu.sync_copy(stage, acc_sh.at[pl.ds(s * RPT, RPT)])
        plsc.subcore_barrier()

        bufs = (b0, b1, b2)
        gsems = (g0, g1, g2)
        ssems = (s0, s1, s2)

        def idx_g(m):
            return sidx.at[pl.ds(m * CHUNK, CHUNK)]

        def idx_s(m):
            return didx.at[pl.ds(m * CHUNK, CHUNK)]

        def fire_g(m, u):
            pltpu.async_copy(table_sh.at[idx_g(m)], bufs[u], gsems[u])

        def wait_g(m, u):
            pltpu.make_async_copy(table_sh.at[idx_g(m)], bufs[u],
                                  gsems[u]).wait()

        def fire_s(m, u):
            pltpu.async_copy(bufs[u], acc_sh.at[idx_s(m)], ssems[u],
                             add=True)

        def wait_s(m, u):
            pltpu.make_async_copy(bufs[u], acc_sh.at[idx_s(m)],
                                  ssems[u]).wait()

        fire_g(0, 0)
        fire_g(1, 1)
        fire_g(2, 2)
        for m in range(CPT32):
            u = m % 3
            wait_g(m, u)
            fire_s(m, u)
            if m + 3 < CPT32:
                wait_s(m, u)
                fire_g(m + 3, u)
        for m in range(CPT32 - 3, CPT32):
            wait_s(m, m % 3)

        plsc.subcore_barrier()
        pltpu.sync_copy(acc_sh.at[pl.ds(s * RPT, RPT)], stage)
        pltpu.sync_copy(stage, out_hbm.at[c, pl.ds(s * RPT, RPT)])

    outs = (jax.ShapeDtypeStruct((2, NROWS, D_HID), jnp.float32),
            jax.ShapeDtypeStruct((NROWS // 8, 128), jnp.float32))
    if not scale:
        outs = (jax.ShapeDtypeStruct((2, NROWS, D_HID), jnp.float32),
                jax.ShapeDtypeStruct((8, 128), jnp.float32))
    return pl.kernel(
        body,
        out_type=outs,
        mesh=_mesh,
        compiler_params=pltpu.CompilerParams(needs_layout_passes=False,
                                             use_tc_tiling_on_sc=False),
        scratch_types=[
            pltpu.VMEM((EPT,), jnp.int32),
            pltpu.VMEM((EPT,), jnp.int32),
            pltpu.VMEM((CHUNK, D_HID), jnp.float32),
            pltpu.VMEM((CHUNK, D_HID), jnp.float32),
            pltpu.VMEM((CHUNK, D_HID), jnp.float32),
            pltpu.VMEM((RPT, D_HID), jnp.float32),
            pltpu.VMEM((RPT // 8, 128), jnp.float32),
            pltpu.VMEM((2, RPT), jnp.float32),
            pltpu.VMEM((16,), jnp.float32),
            pltpu.VMEM_SHARED((NROWS, D_HID), jnp.float32),
            pltpu.VMEM_SHARED((NROWS, D_HID), jnp.float32),
            pltpu.SemaphoreType.DMA,
            pltpu.SemaphoreType.DMA,
            pltpu.SemaphoreType.DMA,
            pltpu.SemaphoreType.DMA,
            pltpu.SemaphoreType.DMA,
            pltpu.SemaphoreType.DMA,
        ],
    )


_agg_scaled_call = _make_agg(True)
_agg_plain_call = _make_agg(False)


def _k2_body(x_ref, w_ref, o_ref):
    o_ref[pl.ds(0, N), :] = jnp.dot(x_ref[...], w_ref[...],
                                    preferred_element_type=jnp.float32,
                                    precision=lax.Precision.HIGHEST)
    o_ref[pl.ds(N, NROWS - N), :] = jnp.zeros((NROWS - N, D_HID), jnp.float32)


def _k4_body(parts_ref, dis_ref, b_ref, o_ref):
    agg = parts_ref[0] + parts_ref[1]
    z = jnp.maximum(agg * dis_ref[...] + b_ref[...], 0.0)
    o_ref[...] = z * dis_ref[...]


def _k6_body(parts_ref, dis_ref, w_ref, b_ref, o_ref):
    u = (parts_ref[0] + parts_ref[1]) * dis_ref[...]
    o = jnp.dot(u, w_ref[...], preferred_element_type=jnp.float32,
                precision=lax.Precision.HIGHEST) + b_ref[...]
    o3 = o.reshape(o.shape[0], 8, D_OUT)
    m = jnp.max(o3, axis=2, keepdims=True)
    e = o3 - m
    lse = jnp.log(jnp.sum(jnp.exp(e), axis=2, keepdims=True))
    o_ref[...] = (e - lse).reshape(o.shape[0], 8 * D_OUT)


NRV = NROWS // 8

_k2_call = pl.pallas_call(
    _k2_body, out_shape=jax.ShapeDtypeStruct((NROWS, D_HID), jnp.float32))

_K4R = 320
_k4_call = pl.pallas_call(
    _k4_body,
    grid=(NRV // _K4R,),
    in_specs=[
        pl.BlockSpec((2, _K4R, 128), lambda i: (0, i, 0)),
        pl.BlockSpec((_K4R, 128), lambda i: (i, 0)),
        pl.BlockSpec((1, 128), lambda i: (0, 0)),
    ],
    out_specs=pl.BlockSpec((_K4R, 128), lambda i: (i, 0)),
    out_shape=jax.ShapeDtypeStruct((NRV, 128), jnp.float32))

_K6R = 320
_k6_call = pl.pallas_call(
    _k6_body,
    grid=(NRV // _K6R,),
    in_specs=[
        pl.BlockSpec((2, _K6R, 128), lambda i: (0, i, 0)),
        pl.BlockSpec((_K6R, 128), lambda i: (i, 0)),
        pl.BlockSpec((128, 8 * D_OUT), lambda i: (0, 0)),
        pl.BlockSpec((1, 8 * D_OUT), lambda i: (0, 0)),
    ],
    out_specs=pl.BlockSpec((_K6R, 8 * D_OUT), lambda i: (i, 0)),
    out_shape=jax.ShapeDtypeStruct((NRV, 8 * D_OUT), jnp.float32))


@jax.jit
def kernel(x, edge_index, W1, b1, W2, b2):
    h1 = _k2_call(x, W1)
    zdeg = jnp.zeros((NROWS,), jnp.float32)
    deg2 = _deg_call(edge_index, zdeg)
    zrows = jnp.zeros((RPT, D_HID), jnp.float32)
    parts1, dis16 = _agg_scaled_call(h1, deg2, edge_index, zrows)
    b1t = jnp.tile(b1, (8,)).reshape(1, 128)
    zp = _k4_call(parts1.reshape(2, NRV, 128), dis16, b1t)
    parts2, _unused = _agg_plain_call(zp.reshape(NROWS, D_HID), deg2,
                                      edge_index, zrows)
    w2rep = jnp.kron(jnp.eye(8, dtype=jnp.float32), W2)
    b2t = jnp.tile(b2, (8,)).reshape(1, 8 * D_OUT)
    out320 = _k6_call(parts2.reshape(2, NRV, 128), dis16, w2rep, b2t)
    return out320[:N // 8].reshape(N, D_OUT)
